# Initial kernel scaffold; baseline (speedup 1.0000x reference)
import jax
import jax.numpy as jnp
from jax.experimental import pallas as pl


def kernel(masks, boxes, feature_map, W3, b3, W1, b1):
    raise NotImplementedError("write your pallas kernel here")


# speedup vs baseline: 7.3050x; 7.3050x over previous
import numpy as np
import jax
import jax.numpy as jnp
from jax.experimental import pallas as pl
from jax.experimental.pallas import tpu as pltpu

_C = 81
_NROI = 100
_MS = 28
_H = _W = 384
_MAXS = 80
_TH = 8
_NS = _H // _TH
_WP = 392
_N = _TH * _WP

_PREC_RESIZE = jax.lax.Precision.HIGHEST
_PREC_CONV = jax.lax.Precision.HIGHEST


def _resize_wmat(n_in: int, n_out: int) -> np.ndarray:
    scale = n_out / n_in
    inv_scale = 1.0 / scale
    kscale = max(inv_scale, 1.0)
    sample_f = (np.arange(n_out, dtype=np.float64) + 0.5) * inv_scale - 0.5
    x = np.abs(sample_f[None, :] - np.arange(n_in, dtype=np.float64)[:, None]) / kscale
    w = np.maximum(0.0, 1.0 - x)
    tot = w.sum(axis=0, keepdims=True)
    w = np.where(np.abs(tot) > 1000.0 * np.finfo(np.float32).eps,
                 w / np.where(tot != 0, tot, 1.0), 0.0)
    ok = (sample_f >= -0.5) & (sample_f <= n_in - 0.5)
    return np.where(ok[None, :], w, 0.0)


def _build_rmat() -> np.ndarray:
    r = np.zeros((_MAXS, _MAXS, _MS), dtype=np.float32)
    for s in range(1, _MAXS + 1):
        r[s - 1, :s, :] = _resize_wmat(_MS, s).T.astype(np.float32)
    return r


_RMAT = _build_rmat()


def _mmap_kernel(boxes_ref, masks_ref, r_ref, m_ref, macc_ref):
    macc_ref[...] = jnp.ones_like(macc_ref)

    def body(t, carry):
        for a in range(4):
            k = 4 * t + a
            x0 = boxes_ref[k, 0]
            y0 = boxes_ref[k, 1]
            w = jnp.maximum(boxes_ref[k, 2] - x0 + 1, 1)
            h = jnp.maximum(boxes_ref[k, 3] - y0 + 1, 1)
            ar = r_ref[w - 1]
            ac = r_ref[h - 1]
            mk = masks_ref[k]
            p = jax.lax.dot(ar, mk, precision=_PREC_RESIZE)
            patch = jax.lax.dot_general(
                p, ac, (((1,), (1,)), ((), ())), precision=_PREC_RESIZE)
            ii = jax.lax.broadcasted_iota(jnp.int32, (_MAXS, _MAXS), 0)
            jj = jax.lax.broadcasted_iota(jnp.int32, (_MAXS, _MAXS), 1)
            f1 = jnp.where((ii < w) & (jj < h), jax.nn.sigmoid(patch), 0.0)
            sj = jax.lax.broadcasted_iota(jnp.int32, (_MAXS, _W), 0)
            sx = jax.lax.broadcasted_iota(jnp.int32, (_MAXS, _W), 1)
            sel = jnp.where(sx - x0 == sj, 1.0, 0.0).astype(jnp.float32)
            strip = jax.lax.dot(f1, sel, precision=_PREC_RESIZE)
            r8 = jax.lax.rem(y0, 8)
            q8 = pl.multiple_of(y0 - r8, 8)
            strip88 = jnp.concatenate(
                [strip, jnp.zeros((8, _W), jnp.float32)], axis=0)
            strip88 = pltpu.roll(strip88, r8, axis=0)
            rows = pl.ds(q8, _MAXS + 8)
            macc_ref[a, rows, :] = macc_ref[a, rows, :] * (1.0 + strip88)
        return carry

    jax.lax.fori_loop(0, _NROI // 4, body, 0)
    m_ref[...] = (macc_ref[0] * macc_ref[1]) * (macc_ref[2] * macc_ref[3])


def _conv_kernel(fm_ref, m_ref, w3_ref, b3_ref, w1_ref, b1_ref, out_ref,
                 xrp_ref, xwin_ref, plast_ref):
    i = pl.program_id(0)
    s = jax.lax.rem(i, 2)

    @pl.when(i == 0)
    def _():
        xrp_ref[...] = jnp.zeros_like(xrp_ref)
        xwin_ref[...] = jnp.zeros_like(xwin_ref)
        plast_ref[...] = jnp.zeros_like(plast_ref)

    xr = jnp.maximum(fm_ref[...] * m_ref[...], 0.0)
    xr = jnp.where(i < _NS, xr, 0.0)
    xrp_ref[s, :, 1:_W + 1, :] = xr

    @pl.when(i >= 1)
    def _():
        prev = xrp_ref[1 - s]
        xwin_ref[0:_WP, :] = plast_ref[...]
        xwin_ref[pl.ds(_WP, _TH * _WP), :] = prev.reshape(_TH * _WP, _C)
        xwin_ref[pl.ds((_TH + 1) * _WP, _WP), :] = xrp_ref[s, 0]
        acc = jnp.zeros((_N, _C), jnp.float32)
        for dy in range(3):
            for dx in range(3):
                off = dy * _WP + dx
                acc = acc + jax.lax.dot(
                    xwin_ref[pl.ds(off, _N), :], w3_ref[3 * dy + dx],
                    precision=_PREC_CONV, preferred_element_type=jnp.float32)
        z = jnp.maximum(acc + b3_ref[0], 0.0)
        y2 = jax.lax.dot(z, w1_ref[...], precision=_PREC_CONV,
                         preferred_element_type=jnp.float32) + b1_ref[0]
        out_ref[...] = y2.reshape(_TH, _WP, _C)[:, 0:_W, :]
        plast_ref[...] = xrp_ref[1 - s, _TH - 1]


def _build_mult(boxes, masks3, rmat):
    return pl.pallas_call(
        _mmap_kernel,
        out_shape=jax.ShapeDtypeStruct((_H, _W), jnp.float32),
        in_specs=[
            pl.BlockSpec(memory_space=pltpu.SMEM),
            pl.BlockSpec(memory_space=pltpu.VMEM),
            pl.BlockSpec(memory_space=pltpu.VMEM),
        ],
        out_specs=pl.BlockSpec(memory_space=pltpu.VMEM),
        scratch_shapes=[pltpu.VMEM((4, _H, _W), jnp.float32)],
    )(boxes, masks3, rmat)


def _fused_conv(fm_hwc, m3, w3t, b3r, w1t, b1r):
    grid = (_NS + 1,)
    return pl.pallas_call(
        _conv_kernel,
        grid=grid,
        out_shape=jax.ShapeDtypeStruct((_H, _W, _C), jnp.float32),
        in_specs=[
            pl.BlockSpec((_TH, _W, _C), lambda i: (jnp.minimum(i, _NS - 1), 0, 0)),
            pl.BlockSpec((_TH, _W, 1), lambda i: (jnp.minimum(i, _NS - 1), 0, 0)),
            pl.BlockSpec((9, _C, _C), lambda i: (0, 0, 0)),
            pl.BlockSpec((1, _C), lambda i: (0, 0)),
            pl.BlockSpec((_C, _C), lambda i: (0, 0)),
            pl.BlockSpec((1, _C), lambda i: (0, 0)),
        ],
        out_specs=pl.BlockSpec((_TH, _W, _C), lambda i: (jnp.maximum(i - 1, 0), 0, 0)),
        scratch_shapes=[
            pltpu.VMEM((2, _TH, _WP, _C), jnp.float32),
            pltpu.VMEM(((_TH + 3) * _WP, _C), jnp.float32),
            pltpu.VMEM((_WP, _C), jnp.float32),
        ],
    )(fm_hwc, m3, w3t, b3r, w1t, b1r)


def kernel(masks, boxes, feature_map, W3, b3, W1, b1):
    boxes = boxes.astype(jnp.int32)
    masks3 = masks.reshape(_NROI, _MS, _MS)
    rmat = jnp.asarray(_RMAT)
    m = _build_mult(boxes, masks3, rmat)
    fm_hwc = feature_map[0].transpose(1, 2, 0)
    m3 = m.reshape(_H, _W, 1)
    w3t = W3.transpose(2, 3, 1, 0).reshape(9, _C, _C)
    w1t = W1[:, :, 0, 0].transpose(1, 0)
    out_hwc = _fused_conv(fm_hwc, m3, w3t, b3.reshape(1, _C), w1t, b1.reshape(1, _C))
    return out_hwc.transpose(2, 0, 1)[None]
